# Initial kernel scaffold; baseline (speedup 1.0000x reference)
import jax
import jax.numpy as jnp
from jax.experimental import pallas as pl


def kernel(x, edge_index, edge_type, W1, b1, Wrel1, Wroot1, brg1, Wrel2, Wroot2, brg2, W2, b2):
    raise NotImplementedError("write your pallas kernel here")


# speedup vs baseline: 11.8525x; 11.8525x over previous
import functools

import jax
import jax.numpy as jnp
from jax import lax
from jax.experimental import pallas as pl
from jax.experimental.pallas import tpu as pltpu
from jax.experimental.pallas import tpu_sc as plsc

N = 10000
E = 320000
D = 128
R = 8
RN = N * R
NC = 2
NS = 16
NW = NC * NS
EPW = E // NW
C = 80
NCH = EPW // C
NP = 10240
RPT = NP // NS
ZR = 32

_mesh = plsc.VectorSubcoreMesh(core_axis_name="c", subcore_axis_name="s")
_sc_params = pltpu.CompilerParams(needs_layout_passes=False)


def _count_body(src_hbm, dst_hbm, typ_hbm, cnt_hbm, gidx_hbm, widx_hbm,
                cntv, sbuf, dbuf, tbuf, gbuf, wbuf):
    cid = lax.axis_index("c")
    sid = lax.axis_index("s")
    wid = sid * NC + cid
    zeros = jnp.zeros((16,), jnp.float32)

    def zero(i, carry):
        for k in range(4):
            cntv[pl.ds(i * 64 + k * 16, 16)] = zeros
        return carry
    lax.fori_loop(0, RN // 64, zero, 0)

    ones = jnp.ones((16,), jnp.float32)

    def chunk(g, carry):
        base = wid * EPW + g * C
        pltpu.sync_copy(src_hbm.at[pl.ds(base, C)], sbuf)
        pltpu.sync_copy(dst_hbm.at[pl.ds(base, C)], dbuf)
        pltpu.sync_copy(typ_hbm.at[pl.ds(base, C)], tbuf)
        for i in range(C // 16):
            sl = pl.ds(i * 16, 16)
            t = tbuf[sl]
            gv = sbuf[sl] * R + t
            wv = dbuf[sl] * R + t
            gbuf[sl] = gv
            wbuf[sl] = wv
            plsc.addupdate_scatter(cntv, [wv], ones)
        pltpu.sync_copy(gbuf, gidx_hbm.at[pl.ds(base, C)])
        pltpu.sync_copy(wbuf, widx_hbm.at[pl.ds(base, C)])
        return carry
    lax.fori_loop(0, NCH, chunk, 0)
    pltpu.sync_copy(cntv, cnt_hbm.at[pl.ds(wid * RN, RN)])


_count_call = functools.partial(
    pl.kernel,
    out_type=(
        jax.ShapeDtypeStruct((NW * RN,), jnp.float32),
        jax.ShapeDtypeStruct((E,), jnp.int32),
        jax.ShapeDtypeStruct((E,), jnp.int32),
    ),
    mesh=_mesh,
    scratch_types=[
        pltpu.VMEM((RN,), jnp.float32),
        pltpu.VMEM((C,), jnp.int32),
        pltpu.VMEM((C,), jnp.int32),
        pltpu.VMEM((C,), jnp.int32),
        pltpu.VMEM((C,), jnp.int32),
        pltpu.VMEM((C,), jnp.int32),
    ],
    compiler_params=_sc_params,
)(_count_body)


CW = 400
NCHW = EPW // CW


def _wgt_body(widx_hbm, recip_hbm, w_hbm, recv, wbuf, wvals):
    cid = lax.axis_index("c")
    sid = lax.axis_index("s")
    wid = sid * NC + cid
    pltpu.sync_copy(recip_hbm, recv)

    def chunk(g, carry):
        base = wid * EPW + g * CW
        pltpu.sync_copy(widx_hbm.at[pl.ds(base, CW)], wbuf)
        for i in range(CW // 16):
            sl = pl.ds(i * 16, 16)
            wvals[sl] = plsc.load_gather(recv, [wbuf[sl]])
        pltpu.sync_copy(wvals, w_hbm.at[pl.ds(base, CW)])
        return carry
    lax.fori_loop(0, NCHW, chunk, 0)


_wgt_call = functools.partial(
    pl.kernel,
    out_type=jax.ShapeDtypeStruct((E,), jnp.float32),
    mesh=_mesh,
    scratch_types=[
        pltpu.VMEM((RN,), jnp.float32),
        pltpu.VMEM((CW,), jnp.int32),
        pltpu.VMEM((CW,), jnp.float32),
    ],
    compiler_params=_sc_params,
)(_wgt_body)


def _agg_body(gidx_hbm, w_hbm, dst_hbm, ptab_hbm, u_hbm,
              gbuf, dbuf, rows, wvals, zbuf, acc, sem):
    cid = lax.axis_index("c")
    sid = lax.axis_index("s")
    wid = sid * NC + cid

    zeros = jnp.zeros((16,), jnp.float32)
    for i in range(ZR):
        for k in range(8):
            zbuf[i, pl.ds(k * 16, 16)] = zeros

    def zero_acc(i, carry):
        pltpu.sync_copy(zbuf, acc.at[pl.ds(sid * RPT + i * ZR, ZR)])
        return carry
    lax.fori_loop(0, RPT // ZR, zero_acc, 0)
    plsc.subcore_barrier()

    def chunk(g, carry):
        base = wid * EPW + g * C
        pltpu.sync_copy(gidx_hbm.at[pl.ds(base, C)], gbuf)
        pltpu.sync_copy(w_hbm.at[pl.ds(base, C)], wvals)
        pltpu.sync_copy(dst_hbm.at[pl.ds(base, C)], dbuf)
        pltpu.async_copy(ptab_hbm.at[gbuf], rows, sem).wait()

        def scale(e, inner):
            w = plsc.load_gather(wvals, [jnp.full((16,), e, jnp.int32)])
            for k in range(8):
                csl = pl.ds(k * 16, 16)
                rows[e, csl] = rows[e, csl] * w
            return inner
        lax.fori_loop(0, C, scale, 0)
        pltpu.sync_copy(rows, acc.at[dbuf], add=True)
        return carry
    lax.fori_loop(0, NCH, chunk, 0)
    plsc.subcore_barrier()
    pltpu.sync_copy(acc.at[pl.ds(sid * RPT, RPT)],
                    u_hbm.at[cid, pl.ds(sid * RPT, RPT)])


_agg_call = functools.partial(
    pl.kernel,
    out_type=jax.ShapeDtypeStruct((NC, NP, D), jnp.float32),
    mesh=_mesh,
    scratch_types=[
        pltpu.VMEM((C,), jnp.int32),
        pltpu.VMEM((C,), jnp.int32),
        pltpu.VMEM((C, D), jnp.float32),
        pltpu.VMEM((C,), jnp.float32),
        pltpu.VMEM((ZR, D), jnp.float32),
        pltpu.VMEM_SHARED((NP, D), jnp.float32),
        pltpu.SemaphoreType.DMA,
    ],
    compiler_params=_sc_params,
)(_agg_body)


NB = 10
BR = N // NB


def _recip_body(cnt_ref, out_ref):
    c = jnp.sum(cnt_ref[...], axis=0)
    out_ref[...] = 1.0 / jnp.maximum(c, 1.0)


def _recip_call(cnt):
    return pl.pallas_call(
        _recip_body,
        out_shape=jax.ShapeDtypeStruct((RN // D, D), jnp.float32),
    )(cnt.reshape(NW, RN // D, D))


def _tca_body(x_ref, w1_ref, b1_ref, wstk_ref, wroot_ref, brg_ref,
              p_ref, root_ref):
    h = jnp.dot(x_ref[...], w1_ref[...].T,
                preferred_element_type=jnp.float32) + b1_ref[...]
    h = jnp.where(h >= 0, h, 0.01 * h)
    p_ref[...] = jnp.dot(h, wstk_ref[...], preferred_element_type=jnp.float32)
    root_ref[...] = jnp.dot(h, wroot_ref[...],
                            preferred_element_type=jnp.float32) + brg_ref[...]


def _tca_call(x, W1, b1, wstk, wroot, brg):
    full = lambda s: pl.BlockSpec(s, lambda i: (0, 0))
    return pl.pallas_call(
        _tca_body,
        grid=(NB,),
        in_specs=[
            pl.BlockSpec((BR, D), lambda i: (i, 0)),
            full((D, D)), full((1, D)), full((D, R * D)), full((D, D)),
            full((1, D)),
        ],
        out_specs=[
            pl.BlockSpec((BR, R * D), lambda i: (i, 0)),
            pl.BlockSpec((BR, D), lambda i: (i, 0)),
        ],
        out_shape=[
            jax.ShapeDtypeStruct((N, R * D), jnp.float32),
            jax.ShapeDtypeStruct((N, D), jnp.float32),
        ],
    )(x, W1, b1.reshape(1, D), wstk, wroot, brg.reshape(1, D))


def _tcb_body(root_ref, u_ref, wstk_ref, wroot_ref, brg_ref,
              p_ref, root2_ref):
    h = root_ref[...] + u_ref[0] + u_ref[1]
    p_ref[...] = jnp.dot(h, wstk_ref[...], preferred_element_type=jnp.float32)
    root2_ref[...] = jnp.dot(h, wroot_ref[...],
                             preferred_element_type=jnp.float32) + brg_ref[...]


def _tcb_call(root, u, wstk, wroot, brg):
    full = lambda s: pl.BlockSpec(s, lambda i: (0, 0))
    return pl.pallas_call(
        _tcb_body,
        grid=(NB,),
        in_specs=[
            pl.BlockSpec((BR, D), lambda i: (i, 0)),
            pl.BlockSpec((NC, BR, D), lambda i: (0, i, 0)),
            full((D, R * D)), full((D, D)), full((1, D)),
        ],
        out_specs=[
            pl.BlockSpec((BR, R * D), lambda i: (i, 0)),
            pl.BlockSpec((BR, D), lambda i: (i, 0)),
        ],
        out_shape=[
            jax.ShapeDtypeStruct((N, R * D), jnp.float32),
            jax.ShapeDtypeStruct((N, D), jnp.float32),
        ],
    )(root, u, wstk, wroot, brg.reshape(1, D))


def _tcc_body(root_ref, u_ref, w2_ref, b2_ref, y_ref):
    h = root_ref[...] + u_ref[0] + u_ref[1]
    y = jnp.dot(h, w2_ref[...].T,
                preferred_element_type=jnp.float32) + b2_ref[...]
    y_ref[...] = jnp.where(y >= 0, y, 0.01 * y)


def _tcc_call(root, u, W2, b2):
    full = lambda s: pl.BlockSpec(s, lambda i: (0, 0))
    return pl.pallas_call(
        _tcc_body,
        grid=(NB,),
        in_specs=[
            pl.BlockSpec((BR, D), lambda i: (i, 0)),
            pl.BlockSpec((NC, BR, D), lambda i: (0, i, 0)),
            full((D, D)), full((1, D)),
        ],
        out_specs=pl.BlockSpec((BR, D), lambda i: (i, 0)),
        out_shape=jax.ShapeDtypeStruct((N, D), jnp.float32),
    )(root, u, W2, b2.reshape(1, D))


def kernel(x, edge_index, edge_type, W1, b1, Wrel1, Wroot1, brg1,
           Wrel2, Wroot2, brg2, W2, b2):
    src = edge_index[0]
    dst = edge_index[1]
    cnt, gidx, widx = _count_call(src, dst, edge_type)
    recip = _recip_call(cnt).reshape(RN)
    w = _wgt_call(widx, recip)
    wstk1 = jnp.transpose(Wrel1, (1, 0, 2)).reshape(D, R * D)
    wstk2 = jnp.transpose(Wrel2, (1, 0, 2)).reshape(D, R * D)
    p1, root1 = _tca_call(x, W1, b1, wstk1, Wroot1, brg1)
    u1 = _agg_call(gidx, w, dst, p1.reshape(RN, D))[:, :N]
    p2, root2 = _tcb_call(root1, u1, wstk2, Wroot2, brg2)
    u2 = _agg_call(gidx, w, dst, p2.reshape(RN, D))[:, :N]
    return _tcc_call(root2, u2, W2, b2)
